# baseline (device time: 91582 ns/iter reference)
import jax
import jax.numpy as jnp
from jax import lax
from jax.experimental import pallas as pl
from jax.experimental.pallas import tpu as pltpu

M = 2048
N = 2048
HALF = M // 2
GCOL = N // 2
WIDTHS = (128, 256, 256, 384)
OFFS = (0, 128, 384, 640)
C = len(WIDTHS)

_MESH = pl.DeviceIdType.MESH


def kernel(dy, W):
    y_idx = lax.axis_index("y")
    zp = lax.axis_index("z") % 2
    dy_half = lax.dynamic_slice_in_dim(dy, y_idx * HALF, HALF, axis=0)
    w_own = lax.dynamic_slice_in_dim(W, zp * GCOL, GCOL, axis=0)
    p = lax.dot_general(
        dy_half, w_own,
        (((1,), (1,)), ((), ())),
        preferred_element_type=jnp.float32,
        precision=lax.Precision.DEFAULT,
    ).astype(jnp.bfloat16)
    return _allreduce_xyz(p)


def _allreduce_xyz(p):
    def body(p_ref, out_ref, xrecv, yrecv, zsend, zrecv,
             xsend_sems, xrecv_sems, ysend_sems, yrecv_sems,
             zsend_sems, zrecv_sems):
        my_x = lax.axis_index("x")
        my_y = lax.axis_index("y")
        my_z = lax.axis_index("z")
        zp = my_z % 2
        x_nbr = (1 - my_x, my_y, my_z)
        y_nbr = (my_x, 1 - my_y, my_z)
        z_nbr = (my_x, my_y, my_z + 1 - 2 * zp)
        row0 = my_y * HALF
        orow0 = (1 - my_y) * HALF
        g0 = zp * GCOL
        og0 = (1 - zp) * GCOL

        barrier = pltpu.get_barrier_semaphore()
        for nbr in (x_nbr, y_nbr, z_nbr):
            pl.semaphore_signal(barrier, inc=1, device_id=nbr,
                                device_id_type=_MESH)
        pl.semaphore_wait(barrier, 3)

        rx, ry, rza, rzb = {}, {}, {}, {}

        def x_desc(c):
            cs = pl.ds(OFFS[c], WIDTHS[c])
            return pltpu.make_async_remote_copy(
                src_ref=p_ref.at[:, cs],
                dst_ref=xrecv.at[:, cs],
                send_sem=xsend_sems.at[c], recv_sem=xrecv_sems.at[c],
                device_id=x_nbr, device_id_type=_MESH)

        def y_desc(c):
            cs = pl.ds(OFFS[c], WIDTHS[c])
            return pltpu.make_async_remote_copy(
                src_ref=zsend.at[pl.ds(row0, HALF), cs],
                dst_ref=yrecv.at[:, cs],
                send_sem=ysend_sems.at[c], recv_sem=yrecv_sems.at[c],
                device_id=y_nbr, device_id_type=_MESH)

        def z_desc(c, r0):
            return pltpu.make_async_remote_copy(
                src_ref=zsend.at[pl.ds(r0, HALF), pl.ds(OFFS[c], WIDTHS[c])],
                dst_ref=zrecv.at[pl.ds(r0, HALF), pl.ds(OFFS[c], WIDTHS[c])],
                send_sem=zsend_sems.at[c], recv_sem=zrecv_sems.at[c],
                device_id=z_nbr, device_id_type=_MESH)

        for i in range(C + 3):
            if i < C:
                c = i
                rx[c] = x_desc(c)
                rx[c].start()
            if 1 <= i <= C:
                c = i - 1
                rx[c].wait_recv()
                cs = pl.ds(OFFS[c], WIDTHS[c])
                s = p_ref[:, cs] + xrecv[:, cs]
                zsend[pl.ds(row0, HALF), cs] = s
                out_ref[pl.ds(row0, HALF), pl.ds(g0 + OFFS[c], WIDTHS[c])] = (
                    s.astype(jnp.float32))
                ry[c] = y_desc(c)
                ry[c].start()
                rza[c] = z_desc(c, row0)
                rza[c].start()
            if 2 <= i <= C + 1:
                c = i - 2
                ry[c].wait_recv()
                cs = pl.ds(OFFS[c], WIDTHS[c])
                yv = yrecv[:, cs]
                zsend[pl.ds(orow0, HALF), cs] = yv
                out_ref[pl.ds(orow0, HALF), pl.ds(g0 + OFFS[c], WIDTHS[c])] = (
                    yv.astype(jnp.float32))
                rzb[c] = z_desc(c, orow0)
                rzb[c].start()
            if 3 <= i <= C + 2:
                c = i - 3
                rza[c].wait_recv()
                rzb[c].wait_recv()
                cs = pl.ds(OFFS[c], WIDTHS[c])
                out_ref[:, pl.ds(og0 + OFFS[c], WIDTHS[c])] = (
                    zrecv[:, cs].astype(jnp.float32))

        for c in range(C):
            rx[c].wait_send()
            ry[c].wait_send()
            rza[c].wait_send()
            rzb[c].wait_send()

    return pl.pallas_call(
        body,
        out_shape=jax.ShapeDtypeStruct((M, N), jnp.float32),
        in_specs=[pl.BlockSpec(memory_space=pltpu.VMEM)],
        out_specs=pl.BlockSpec(memory_space=pltpu.VMEM),
        scratch_shapes=[
            pltpu.VMEM((HALF, GCOL), jnp.bfloat16),
            pltpu.VMEM((HALF, GCOL), jnp.bfloat16),
            pltpu.VMEM((M, GCOL), jnp.bfloat16),
            pltpu.VMEM((M, GCOL), jnp.bfloat16),
            pltpu.SemaphoreType.DMA((C,)),
            pltpu.SemaphoreType.DMA((C,)),
            pltpu.SemaphoreType.DMA((C,)),
            pltpu.SemaphoreType.DMA((C,)),
            pltpu.SemaphoreType.DMA((C,)),
            pltpu.SemaphoreType.DMA((C,)),
        ],
        compiler_params=pltpu.CompilerParams(collective_id=0),
    )(p)
